# baseline (speedup 1.0000x reference)
import functools

import jax
import jax.numpy as jnp
from jax import lax
from jax.experimental import pallas as pl
from jax.experimental.pallas import tpu as pltpu
from jax.experimental.pallas import tpu_sc as plsc

NC = 2
NS = 16
L = 16
NW = NC * NS

D = 64
CHUNK = 512


def kernel(t, emb):
    A, B, Dp = t.shape
    assert Dp == D
    V, De = emb.shape
    assert De == D
    R = A * B
    assert R % NW == 0
    rows_per_w = R // NW
    assert rows_per_w % CHUNK == 0
    n_chunks = rows_per_w // CHUNK

    t2 = t.reshape(R, D)
    mesh = plsc.VectorSubcoreMesh(core_axis_name="c", subcore_axis_name="s")

    @functools.partial(
        pl.kernel,
        mesh=mesh,
        compiler_params=pltpu.CompilerParams(
            use_tc_tiling_on_sc=False, needs_layout_passes=False
        ),
        out_type=jax.ShapeDtypeStruct((R, 2 * D), jnp.float32),
        scratch_types=[
            pltpu.VMEM((CHUNK, D), jnp.float32),
            pltpu.VMEM((CHUNK, D), jnp.float32),
            pltpu.VMEM((CHUNK,), jnp.int32),
            pltpu.SemaphoreType.DMA,
        ],
    )
    def body(t_hbm, emb_hbm, out_hbm, tbuf, ebuf, iv, sem):
        wid = lax.axis_index("s") * NC + lax.axis_index("c")
        wbase = wid * rows_per_w

        lane = lax.iota(jnp.int32, L)
        col2 = jnp.full((L,), 2, jnp.int32)
        vmax = jnp.full((L,), V - 1, jnp.int32)
        zero = jnp.zeros((L,), jnp.int32)

        def chunk_body(ci, carry):
            r0 = wbase + ci * CHUNK
            pltpu.sync_copy(t_hbm.at[pl.ds(r0, CHUNK)], tbuf)

            def group_body(j, carry2):
                rows = lane + j * L
                vals = plsc.load_gather(tbuf, [rows, col2])
                idx = jnp.minimum(jnp.maximum(vals.astype(jnp.int32), zero), vmax)
                iv[pl.ds(j * L, L)] = idx
                return carry2

            lax.fori_loop(0, CHUNK // L, group_body, 0)

            pltpu.async_copy(emb_hbm.at[iv], ebuf, sem).wait()
            pltpu.sync_copy(tbuf, out_hbm.at[pl.ds(r0, CHUNK), pl.ds(0, D)])
            pltpu.sync_copy(ebuf, out_hbm.at[pl.ds(r0, CHUNK), pl.ds(D, D)])
            return carry

        lax.fori_loop(0, n_chunks, chunk_body, 0)

    out2 = body(t2, emb)
    return out2.reshape(A, B, 2 * D)


# speedup vs baseline: 1.0001x; 1.0001x over previous
import functools

import jax
import jax.numpy as jnp
from jax import lax
from jax.experimental import pallas as pl
from jax.experimental.pallas import tpu as pltpu
from jax.experimental.pallas import tpu_sc as plsc

NC = 2
NS = 16
L = 16
NW = NC * NS

D = 64
CHUNK = 256
NBUF = 2


def kernel(t, emb):
    A, B, Dp = t.shape
    assert Dp == D
    V, De = emb.shape
    assert De == D
    R = A * B
    assert R % NW == 0
    rows_per_w = R // NW
    assert rows_per_w % (CHUNK * NBUF) == 0
    n_chunks = rows_per_w // CHUNK
    n_groups = n_chunks // NBUF

    t2 = t.reshape(R, D)
    mesh = plsc.VectorSubcoreMesh(core_axis_name="c", subcore_axis_name="s")

    @functools.partial(
        pl.kernel,
        mesh=mesh,
        compiler_params=pltpu.CompilerParams(
            use_tc_tiling_on_sc=False, needs_layout_passes=False
        ),
        out_type=jax.ShapeDtypeStruct((R, 2 * D), jnp.float32),
        scratch_types=[
            pltpu.VMEM((NBUF, CHUNK, D), jnp.float32),
            pltpu.VMEM((NBUF, CHUNK, D), jnp.float32),
            pltpu.VMEM((NBUF, CHUNK), jnp.int32),
            pltpu.SemaphoreType.DMA,
            pltpu.SemaphoreType.DMA,
            pltpu.SemaphoreType.DMA,
            pltpu.SemaphoreType.DMA,
            pltpu.SemaphoreType.DMA,
            pltpu.SemaphoreType.DMA,
            pltpu.SemaphoreType.DMA,
            pltpu.SemaphoreType.DMA,
        ],
    )
    def body(t_hbm, emb_hbm, out_hbm, tbuf, ebuf, iv,
             isem0, isem1, tosem0, tosem1, eosem0, eosem1, gsem0, gsem1):
        isem = [isem0, isem1]
        tosem = [tosem0, tosem1]
        eosem = [eosem0, eosem1]
        gsem = [gsem0, gsem1]

        wid = lax.axis_index("s") * NC + lax.axis_index("c")
        wbase = wid * rows_per_w

        lane = lax.iota(jnp.int32, L)
        col2 = jnp.full((L,), 2, jnp.int32)
        vmax = jnp.full((L,), V - 1, jnp.int32)
        zero = jnp.zeros((L,), jnp.int32)

        def in_copy(ci, b):
            return pltpu.make_async_copy(
                t_hbm.at[pl.ds(wbase + ci * CHUNK, CHUNK)], tbuf.at[b], isem[b]
            )

        def tout_copy(ci, b):
            return pltpu.make_async_copy(
                tbuf.at[b],
                out_hbm.at[pl.ds(wbase + ci * CHUNK, CHUNK), pl.ds(0, D)],
                tosem[b],
            )

        def gather_copy(b):
            return pltpu.make_async_copy(emb_hbm.at[iv.at[b]], ebuf.at[b], gsem[b])

        def eout_copy(ci, b):
            return pltpu.make_async_copy(
                ebuf.at[b],
                out_hbm.at[pl.ds(wbase + ci * CHUNK, CHUNK), pl.ds(D, D)],
                eosem[b],
            )

        in_copy(0, 0).start()
        in_copy(1, 1).start()

        def group_body(g, carry):
            for b in range(NBUF):
                ci = g * NBUF + b

                in_copy(ci, b).wait()
                tout_copy(ci, b).start()

                def idx_body(j, carry2):
                    rows = lane + j * L
                    vals = plsc.load_gather(tbuf.at[b], [rows, col2])
                    idx = jnp.minimum(
                        jnp.maximum(vals.astype(jnp.int32), zero), vmax
                    )
                    iv[b, pl.ds(j * L, L)] = idx
                    return carry2

                lax.fori_loop(0, CHUNK // L, idx_body, 0)

                @pl.when(g >= 1)
                def _():
                    eout_copy(ci - NBUF, b).wait()

                gather_copy(b).start()

                pb = 1 - b
                if b == 0:
                    @pl.when(g >= 1)
                    def _():
                        gather_copy(pb).wait()
                        eout_copy(ci - 1, pb).start()
                else:
                    gather_copy(pb).wait()
                    eout_copy(ci - 1, pb).start()

                tout_copy(ci, b).wait()

                @pl.when(g < n_groups - 1)
                def _():
                    in_copy(ci + NBUF, b).start()
            return carry

        lax.fori_loop(0, n_groups, group_body, 0)

        last = n_chunks - 1
        gather_copy(1).wait()
        eout_copy(last, 1).start()
        eout_copy(last - 1, 0).wait()
        eout_copy(last, 1).wait()

    out2 = body(t2, emb)
    return out2.reshape(A, B, 2 * D)
